# Initial kernel scaffold; baseline (speedup 1.0000x reference)
import jax
import jax.numpy as jnp
from jax.experimental import pallas as pl


def kernel(v_feat, t_feat, user_emb, item_emb, Wi, bi, Wt, bt, q1W, q1b, q2W, edge_index, img_orig_rows, img_orig_cols, img_orig_vals, txt_orig_rows, txt_orig_cols, txt_orig_vals):
    raise NotImplementedError("write your pallas kernel here")


# speedup vs baseline: 1.0954x; 1.0954x over previous
import jax
import jax.numpy as jnp
from jax.experimental import pallas as pl
from jax.experimental.pallas import tpu as pltpu

NUM_USER = 16384
NUM_ITEM = 4096
DIM_E = 64
FEAT_DIM = 128
TOPK = 10
LAMBDA = 0.9


def _mm_bias_body(x_ref, w_ref, b_ref, o_ref):
    o_ref[...] = jnp.dot(x_ref[...], w_ref[...],
                         preferred_element_type=jnp.float32) + b_ref[...]


def _mm_bias(x, w, b):
    return pl.pallas_call(
        _mm_bias_body,
        out_shape=jax.ShapeDtypeStruct((x.shape[0], w.shape[1]), jnp.float32),
    )(x, w, b.reshape(1, -1))


def _build_knn_sym(feats, topk, n):
    fn = feats / jnp.linalg.norm(feats, axis=-1, keepdims=True)
    sim = fn @ fn.T
    knn_val, knn_ind = jax.lax.top_k(sim, topk)
    rows = jnp.repeat(jnp.arange(n, dtype=jnp.int32), topk)
    cols = knn_ind.reshape(-1).astype(jnp.int32)
    vals = knn_val.reshape(-1)
    deg = jax.ops.segment_sum(vals, rows, num_segments=n)
    dis = jnp.where(deg > 0, deg ** -0.5, 0.0)
    vals = dis[rows] * vals * dis[cols]
    return rows, cols, vals


def _spmm(rows, cols, vals, x, n):
    return jax.ops.segment_sum(vals[:, None] * x[cols], rows, num_segments=n)


def _gcn(x, edge_index):
    row, col = edge_index[0], edge_index[1]
    n = x.shape[0]
    deg = jax.ops.segment_sum(jnp.ones(row.shape, x.dtype), row, num_segments=n)
    dis = jnp.where(deg > 0, deg ** -0.5, 0.0)
    norm = dis[row] * dis[col]
    return jax.ops.segment_sum(norm[:, None] * x[row], col, num_segments=n)


def kernel(v_feat, t_feat, user_emb, item_emb, Wi, bi, Wt, bt, q1W, q1b, q2W,
           edge_index, img_orig_rows, img_orig_cols, img_orig_vals,
           txt_orig_rows, txt_orig_cols, txt_orig_vals):
    image_feats = _mm_bias(v_feat, Wi, bi)
    text_feats = _mm_bias(t_feat, Wt, bt)
    ir, ic, iv = _build_knn_sym(image_feats, TOPK, NUM_ITEM)
    rows_i = jnp.concatenate([ir, img_orig_rows])
    cols_i = jnp.concatenate([ic, img_orig_cols])
    vals_i = jnp.concatenate([(1.0 - LAMBDA) * iv, LAMBDA * img_orig_vals])
    tr, tc, tv = _build_knn_sym(text_feats, TOPK, NUM_ITEM)
    rows_t = jnp.concatenate([tr, txt_orig_rows])
    cols_t = jnp.concatenate([tc, txt_orig_cols])
    vals_t = jnp.concatenate([(1.0 - LAMBDA) * tv, LAMBDA * txt_orig_vals])
    image_item_embeds = _spmm(rows_i, cols_i, vals_i, item_emb, NUM_ITEM)
    text_item_embeds = _spmm(rows_t, cols_t, vals_t, item_emb, NUM_ITEM)

    def query(x):
        return jnp.tanh(x @ q1W + q1b) @ q2W

    att = jnp.concatenate([query(image_item_embeds), query(text_item_embeds)], axis=-1)
    weight = jax.nn.softmax(att, axis=-1)
    h = weight[:, 0:1] * image_item_embeds + weight[:, 1:2] * text_item_embeds
    ego = jnp.concatenate([user_emb, item_emb], axis=0)
    ei = jnp.concatenate([edge_index, edge_index[::-1]], axis=1)
    all_embs = [ego]
    for _ in range(2):
        ego = _gcn(ego, ei)
        all_embs.append(ego)
    all_emb = jnp.mean(jnp.stack(all_embs, axis=1), axis=1)
    return all_emb, h


# speedup vs baseline: 4.5693x; 4.1713x over previous
import functools

import jax
import jax.numpy as jnp
from jax import lax
from jax.experimental import pallas as pl
from jax.experimental.pallas import tpu as pltpu

NUM_USER = 16384
NUM_ITEM = 4096
NUM_NODE = NUM_USER + NUM_ITEM
DIM_E = 64
FEAT_DIM = 128
TOPK = 10
LAMBDA = 0.9
N_INTER = 262144
RB = 256
NBLK = NUM_ITEM // RB


def _feat_body(v_ref, wi_ref, bi_ref, t_ref, wt_ref, bt_ref, fi_ref, ft_ref):
    xi = jnp.dot(v_ref[...], wi_ref[...], preferred_element_type=jnp.float32)
    xi = xi + bi_ref[...]
    ni = jnp.sqrt(jnp.sum(xi * xi, axis=1, keepdims=True))
    fi_ref[...] = xi / ni
    xt = jnp.dot(t_ref[...], wt_ref[...], preferred_element_type=jnp.float32)
    xt = xt + bt_ref[...]
    nt = jnp.sqrt(jnp.sum(xt * xt, axis=1, keepdims=True))
    ft_ref[...] = xt / nt


def _feat_normalize(v_feat, Wi, bi, t_feat, Wt, bt):
    return pl.pallas_call(
        _feat_body,
        out_shape=(jax.ShapeDtypeStruct((NUM_ITEM, DIM_E), jnp.float32),
                   jax.ShapeDtypeStruct((NUM_ITEM, DIM_E), jnp.float32)),
    )(v_feat, Wi, bi.reshape(1, -1), t_feat, Wt, bt.reshape(1, -1))


def _topk_body(fnb_ref, fn_ref, thr_ref, dis_ref):
    sim = lax.dot_general(fnb_ref[0], fn_ref[0],
                          (((1,), (1,)), ((), ())),
                          preferred_element_type=jnp.float32)
    s = sim
    deg = jnp.zeros((RB, 1), jnp.float32)
    thr = jnp.zeros((RB, 1), jnp.float32)
    for _ in range(TOPK):
        m = jnp.max(s, axis=1, keepdims=True)
        deg = deg + m
        thr = m
        s = jnp.where(s == m, -jnp.inf, s)
    dis = jnp.where(deg > 0, lax.rsqrt(deg), 0.0)
    thr_ref[...] = jnp.broadcast_to(thr, (RB, 128))[None]
    dis_ref[...] = jnp.broadcast_to(dis, (RB, 128))[None]


def _topk_thr_dis(fns):
    return pl.pallas_call(
        _topk_body,
        grid=(2, NBLK),
        in_specs=[
            pl.BlockSpec((1, RB, DIM_E), lambda m, b: (m, b, 0)),
            pl.BlockSpec((1, NUM_ITEM, DIM_E), lambda m, b: (m, 0, 0)),
        ],
        out_specs=(
            pl.BlockSpec((1, RB, 128), lambda m, b: (m, b, 0)),
            pl.BlockSpec((1, RB, 128), lambda m, b: (m, b, 0)),
        ),
        out_shape=(jax.ShapeDtypeStruct((2, NUM_ITEM, 128), jnp.float32),
                   jax.ShapeDtypeStruct((2, NUM_ITEM, 128), jnp.float32)),
    )(fns, fns)


def _newspmm_body(fnb_ref, fn_ref, thr_ref, disr_ref, disc_ref, emb_ref, o_ref):
    sim = lax.dot_general(fnb_ref[0], fn_ref[0],
                          (((1,), (1,)), ((), ())),
                          preferred_element_type=jnp.float32)
    thr = thr_ref[0, :, 0:1]
    disr = disr_ref[0, :, 0:1]
    disc = disc_ref[0]
    w = jnp.where(sim >= thr, sim, 0.0)
    w = w * ((1.0 - LAMBDA) * disr) * disc
    o_ref[...] = jnp.dot(w, emb_ref[...],
                         preferred_element_type=jnp.float32)[None]


def _new_spmm(fns, thr, dis, dis_row, item_emb):
    return pl.pallas_call(
        _newspmm_body,
        grid=(2, NBLK),
        in_specs=[
            pl.BlockSpec((1, RB, DIM_E), lambda m, b: (m, b, 0)),
            pl.BlockSpec((1, NUM_ITEM, DIM_E), lambda m, b: (m, 0, 0)),
            pl.BlockSpec((1, RB, 128), lambda m, b: (m, b, 0)),
            pl.BlockSpec((1, RB, 128), lambda m, b: (m, b, 0)),
            pl.BlockSpec((1, 1, NUM_ITEM), lambda m, b: (m, 0, 0)),
            pl.BlockSpec((NUM_ITEM, DIM_E), lambda m, b: (0, 0)),
        ],
        out_specs=pl.BlockSpec((1, RB, DIM_E), lambda m, b: (m, b, 0)),
        out_shape=jax.ShapeDtypeStruct((2, NUM_ITEM, DIM_E), jnp.float32),
    )(fns, fns, thr, dis, dis_row, item_emb)


def _att_body(new_ref, orig_ref, q1w_ref, q1b_ref, q2w_ref, h_ref):
    ii = new_ref[0] + LAMBDA * orig_ref[0]
    tt = new_ref[1] + LAMBDA * orig_ref[1]
    q1w = q1w_ref[...]
    q1b = q1b_ref[...]
    q2w = q2w_ref[...]

    def att(x):
        t = jnp.tanh(jnp.dot(x, q1w, preferred_element_type=jnp.float32) + q1b)
        return jnp.sum(t * q2w, axis=1, keepdims=True)

    a1 = att(ii)
    a2 = att(tt)
    m = jnp.maximum(a1, a2)
    e1 = jnp.exp(a1 - m)
    e2 = jnp.exp(a2 - m)
    inv = 1.0 / (e1 + e2)
    h_ref[...] = (e1 * inv) * ii + (e2 * inv) * tt


def _attention_h(new_spmm, orig_spmm, q1W, q1b, q2W):
    return pl.pallas_call(
        _att_body,
        out_shape=jax.ShapeDtypeStruct((NUM_ITEM, DIM_E), jnp.float32),
    )(new_spmm, orig_spmm, q1W, q1b.reshape(1, -1), q2W.reshape(1, -1))


def _scale_body(x_ref, dis_ref, o_ref):
    o_ref[...] = x_ref[...] * dis_ref[...]


_GB = 2048


def _scale(x, dis):
    return pl.pallas_call(
        _scale_body,
        grid=(NUM_NODE // _GB,),
        in_specs=[pl.BlockSpec((_GB, DIM_E), lambda i: (i, 0)),
                  pl.BlockSpec((_GB, 1), lambda i: (i, 0))],
        out_specs=pl.BlockSpec((_GB, DIM_E), lambda i: (i, 0)),
        out_shape=jax.ShapeDtypeStruct(x.shape, jnp.float32),
    )(x, dis)


def _combine_body(p_ref, dis_ref, x_ref, xs_ref):
    d = dis_ref[...]
    x = (p_ref[0] + p_ref[1]) * d
    x_ref[...] = x
    xs_ref[...] = x * d


def _combine_scale(partials, dis):
    return pl.pallas_call(
        _combine_body,
        grid=(NUM_NODE // _GB,),
        in_specs=[pl.BlockSpec((2, _GB, DIM_E), lambda i: (0, i, 0)),
                  pl.BlockSpec((_GB, 1), lambda i: (i, 0))],
        out_specs=(pl.BlockSpec((_GB, DIM_E), lambda i: (i, 0)),
                   pl.BlockSpec((_GB, DIM_E), lambda i: (i, 0))),
        out_shape=(jax.ShapeDtypeStruct((NUM_NODE, DIM_E), jnp.float32),
                   jax.ShapeDtypeStruct((NUM_NODE, DIM_E), jnp.float32)),
    )(partials, dis)


def _final_body(ego_ref, x1_ref, p_ref, dis_ref, o_ref):
    x2 = (p_ref[0] + p_ref[1]) * dis_ref[...]
    o_ref[...] = (ego_ref[...] + x1_ref[...] + x2) * (1.0 / 3.0)


def _final_mean(ego, x1, partials, dis):
    return pl.pallas_call(
        _final_body,
        grid=(NUM_NODE // _GB,),
        in_specs=[pl.BlockSpec((_GB, DIM_E), lambda i: (i, 0)),
                  pl.BlockSpec((_GB, DIM_E), lambda i: (i, 0)),
                  pl.BlockSpec((2, _GB, DIM_E), lambda i: (0, i, 0)),
                  pl.BlockSpec((_GB, 1), lambda i: (i, 0))],
        out_specs=pl.BlockSpec((_GB, DIM_E), lambda i: (i, 0)),
        out_shape=jax.ShapeDtypeStruct((NUM_NODE, DIM_E), jnp.float32),
    )(ego, x1, partials, dis)


def _deg_fallback(edge_index):
    ones = jnp.ones((N_INTER,), jnp.float32)
    du = jax.ops.segment_sum(ones, edge_index[0], num_segments=NUM_NODE)
    di = jax.ops.segment_sum(ones, edge_index[1], num_segments=NUM_NODE)
    return du + di


def _gcn_layer_fallback(xs, edge_index):
    u, i = edge_index[0], edge_index[1]
    p0 = jax.ops.segment_sum(xs[u], i, num_segments=NUM_NODE)
    p1 = jax.ops.segment_sum(xs[i], u, num_segments=NUM_NODE)
    return jnp.stack([p0, p1], axis=0)


def _orig_spmm_fallback(cols_img, vals_img, cols_txt, vals_txt, item_emb):
    oi = jax.ops.segment_sum(
        vals_img[:, None] * item_emb[cols_img],
        jnp.repeat(jnp.arange(NUM_ITEM, dtype=jnp.int32), TOPK),
        num_segments=NUM_ITEM)
    ot = jax.ops.segment_sum(
        vals_txt[:, None] * item_emb[cols_txt],
        jnp.repeat(jnp.arange(NUM_ITEM, dtype=jnp.int32), TOPK),
        num_segments=NUM_ITEM)
    return jnp.stack([oi, ot], axis=0)


def kernel(v_feat, t_feat, user_emb, item_emb, Wi, bi, Wt, bt, q1W, q1b, q2W,
           edge_index, img_orig_rows, img_orig_cols, img_orig_vals,
           txt_orig_rows, txt_orig_cols, txt_orig_vals):
    fi, ft = _feat_normalize(v_feat, Wi, bi, t_feat, Wt, bt)
    fns = jnp.stack([fi, ft], axis=0)
    thr, dis = _topk_thr_dis(fns)
    dis_row = dis[:, :, 0].reshape(2, 1, NUM_ITEM)
    new_spmm = _new_spmm(fns, thr, dis, dis_row, item_emb)
    orig_spmm = _orig_spmm_fallback(img_orig_cols, img_orig_vals,
                                    txt_orig_cols, txt_orig_vals, item_emb)
    h = _attention_h(new_spmm, orig_spmm, q1W, q1b, q2W)

    ego = jnp.concatenate([user_emb, item_emb], axis=0)
    deg = _deg_fallback(edge_index)
    gdis = jnp.where(deg > 0, deg ** -0.5, 0.0).reshape(NUM_NODE, 1)
    xs0 = _scale(ego, gdis)
    p1 = _gcn_layer_fallback(xs0, edge_index)
    x1, xs1 = _combine_scale(p1, gdis)
    p2 = _gcn_layer_fallback(xs1, edge_index)
    all_emb = _final_mean(ego, x1, p2, gdis)
    return all_emb, h
